# baseline (speedup 1.0000x reference)
import functools

import jax
import jax.numpy as jnp
from jax import lax
from jax.experimental import pallas as pl
from jax.experimental.pallas import tpu as pltpu
from jax.experimental.pallas import tpu_sc as plsc

N = 50000
NPAD = 50048
F = 16
W = 24
NC = 2
NS = 16
NW = NC * NS
B = 1000
BPAD = 1008
Q = 40
SUBS = B // Q
ROWS_PER_TILE = NPAD // NS


def _sc_segment_accumulate(coordT, idx2d, weights, maskc):
    E = weights.shape[0]
    EP = E // NW
    NCHUNK = EP // B

    mesh = plsc.VectorSubcoreMesh(core_axis_name="c", subcore_axis_name="s")

    @functools.partial(
        pl.kernel,
        mesh=mesh,
        compiler_params=pltpu.CompilerParams(use_tc_tiling_on_sc=False),
        out_type=jax.ShapeDtypeStruct((NC, NPAD, W), jnp.float32),
        scratch_types=[
            pltpu.VMEM((SUBS, Q), jnp.int32),
            pltpu.VMEM((BPAD,), jnp.float32),
            pltpu.VMEM((F, BPAD), jnp.float32),
            pltpu.VMEM((BPAD, W), jnp.float32),
            pltpu.VMEM((8, 16), jnp.float32),
            pltpu.VMEM_SHARED((NPAD, W), jnp.float32),
            pltpu.SemaphoreType.DMA,
            pltpu.SemaphoreType.DMA,
        ],
    )
    def sc_kernel(ct_hbm, idx_hbm, w_hbm, mask_hbm, out_hbm,
                  idx_v, w_v, ct_v, wrow_v, mask_v, acc, sem_in, sem_sc):
        c = lax.axis_index("c")
        s = lax.axis_index("s")
        wid = c * NS + s

        lanes = lax.iota(jnp.int32, 16)
        pltpu.sync_copy(mask_hbm, mask_v)
        xidx = {d: lanes ^ d for d in (1, 2, 4, 8)}
        mF = {d: mask_v[k, :] for k, d in enumerate((1, 2, 4, 8))}
        nmF = {d: mask_v[4 + k, :] for k, d in enumerate((1, 2, 4, 8))}

        def zrow(i, carry):
            wrow_v[i, pl.ds(0, 16)] = jnp.zeros((16,), jnp.float32)
            wrow_v[i, pl.ds(W - 16, 16)] = jnp.zeros((16,), jnp.float32)
            return carry
        lax.fori_loop(0, BPAD, zrow, 0)
        r0 = s * ROWS_PER_TILE
        for off, sz in ((0, 1000), (1000, 1000), (2000, 1000), (3000, 128)):
            pltpu.sync_copy(wrow_v.at[pl.ds(0, sz), :],
                            acc.at[pl.ds(r0 + off, sz), :])
        plsc.subcore_barrier()

        def chunk_body(ci, carry):
            base = wid * EP + ci * B
            cps = [pltpu.async_copy(idx_hbm.at[pl.ds(base // Q, SUBS), :],
                                    idx_v, sem_in),
                   pltpu.async_copy(w_hbm.at[pl.ds(base, B)],
                                    w_v.at[pl.ds(0, B)], sem_in)]
            for f in range(F):
                cps.append(pltpu.async_copy(
                    ct_hbm.at[f, pl.ds(base, B)],
                    ct_v.at[f, pl.ds(0, B)], sem_in))
            for cp in cps:
                cp.wait()

            def g_body(g, carry2):
                e0 = g * 16
                wvec = w_v[pl.ds(e0, 16)]
                R = [ct_v[f, pl.ds(e0, 16)] for f in range(F)]
                for d in (1, 2, 4, 8):
                    m, nm, xi = mF[d], nmF[d], xidx[d]
                    newR = list(R)
                    for i in range(16):
                        if i & d:
                            continue
                        A, Bv = R[i], R[i | d]
                        Ash = A.at[xi].get(mode="promise_in_bounds")
                        Bsh = Bv.at[xi].get(mode="promise_in_bounds")
                        newR[i] = A * nm + Bsh * m
                        newR[i | d] = Bv * m + Ash * nm
                    R = newR
                for j in range(16):
                    e = e0 + j
                    wrow_v[e, pl.ds(W - 16, 16)] = jnp.full(
                        (16,), wvec[j], jnp.float32)
                    wrow_v[e, pl.ds(0, F)] = R[j]
                return carry2
            lax.fori_loop(0, BPAD // 16, g_body, 0)

            scs = []
            for j in range(SUBS):
                scs.append(pltpu.async_copy(
                    wrow_v.at[pl.ds(j * Q, Q), :],
                    acc.at[idx_v.at[j]], sem_sc, add=True))
            for d in scs:
                d.wait()
            return carry
        lax.fori_loop(0, NCHUNK, chunk_body, 0)

        plsc.subcore_barrier()
        pltpu.sync_copy(acc.at[pl.ds(r0, ROWS_PER_TILE), :],
                        out_hbm.at[c, pl.ds(r0, ROWS_PER_TILE), :])

    return sc_kernel(coordT, idx2d, weights, maskc)


def _tc_finish_body(p_ref, emb_ref, o_ref):
    a = p_ref[0] + p_ref[1]
    feat = a[:, 0:F]
    ws = a[:, F:F + 1]
    y = lax.dot_general(feat, emb_ref[...],
                        (((1,), (1,)), ((), ())),
                        preferred_element_type=jnp.float32)
    o_ref[...] = y / ws


def _tc_finish(partials, embedding):
    U = embedding.shape[0]
    R = 2000
    grid = (N // R,)
    return pl.pallas_call(
        _tc_finish_body,
        grid=grid,
        in_specs=[
            pl.BlockSpec((NC, R, W), lambda i: (0, i, 0)),
            pl.BlockSpec((U, F), lambda i: (0, 0)),
        ],
        out_specs=pl.BlockSpec((R, U), lambda i: (i, 0)),
        out_shape=jax.ShapeDtypeStruct((N, U), jnp.float32),
    )(partials, embedding)


def _butterfly_masks():
    import numpy as np
    lanes = np.arange(16)
    rows = [((lanes // d) % 2).astype(np.float32) for d in (1, 2, 4, 8)]
    rows += [1.0 - r for r in rows]
    return jnp.asarray(np.stack(rows))


def kernel(coord_features, indices, weights, embedding):
    coordT = coord_features.T * weights[None, :]
    idx2d = indices.reshape(indices.shape[0] // Q, Q)
    partials = _sc_segment_accumulate(coordT, idx2d, weights,
                                      _butterfly_masks())
    return _tc_finish(partials, embedding)


# speedup vs baseline: 1.9870x; 1.9870x over previous
import functools

import jax
import jax.numpy as jnp
from jax import lax
from jax.experimental import pallas as pl
from jax.experimental.pallas import tpu as pltpu
from jax.experimental.pallas import tpu_sc as plsc

N = 50000
NPAD = 50048
F = 16
W = 24
NC = 2
NS = 16
NW = NC * NS
B = 1000
BPAD = 1008
Q = 40
SUBS = B // Q
ROWS_PER_TILE = NPAD // NS


def _sc_segment_accumulate(coord_features, indices, weights):
    E = indices.shape[0]
    EP = E // NW
    NCHUNK = EP // B

    mesh = plsc.VectorSubcoreMesh(core_axis_name="c", subcore_axis_name="s")

    @functools.partial(
        pl.kernel,
        mesh=mesh,
        compiler_params=pltpu.CompilerParams(use_tc_tiling_on_sc=False),
        out_type=jax.ShapeDtypeStruct((NC, NPAD, W), jnp.float32),
        scratch_types=[
            pltpu.VMEM((SUBS, Q), jnp.int32),
            pltpu.VMEM((BPAD,), jnp.float32),
            pltpu.VMEM((BPAD, F), jnp.float32),
            pltpu.VMEM((BPAD, W), jnp.float32),
            pltpu.VMEM_SHARED((NPAD, W), jnp.float32),
            pltpu.SemaphoreType.DMA,
            pltpu.SemaphoreType.DMA,
        ],
    )
    def sc_kernel(coord_hbm, idx_hbm, w_hbm, out_hbm,
                  idx_v, w_v, coord_v, wrow_v, acc, sem, sem_sc):
        c = lax.axis_index("c")
        s = lax.axis_index("s")
        wid = c * NS + s

        def zrow(i, carry):
            wrow_v[i, pl.ds(0, 16)] = jnp.zeros((16,), jnp.float32)
            wrow_v[i, pl.ds(W - 16, 16)] = jnp.zeros((16,), jnp.float32)
            return carry
        lax.fori_loop(0, BPAD, zrow, 0)

        r0 = s * ROWS_PER_TILE
        for off, sz in ((0, 1000), (1000, 1000), (2000, 1000), (3000, 128)):
            pltpu.sync_copy(wrow_v.at[pl.ds(0, sz), :],
                            acc.at[pl.ds(r0 + off, sz), :])
        plsc.subcore_barrier()

        def chunk_body(ci, carry):
            base = wid * EP + ci * B
            copies = [
                pltpu.make_async_copy(
                    w_hbm.at[pl.ds(base, B)], w_v.at[pl.ds(0, B)], sem),
                pltpu.make_async_copy(
                    coord_hbm.at[pl.ds(base, B), :],
                    coord_v.at[pl.ds(0, B), :], sem),
                pltpu.make_async_copy(
                    idx_hbm.at[pl.ds(base // Q, SUBS), :], idx_v, sem),
            ]
            for cp in copies:
                cp.start()
            for cp in copies:
                cp.wait()

            def g_body(g, carry2):
                e0 = g * 16
                wvec = w_v[pl.ds(e0, 16)]
                for j in range(16):
                    e = e0 + j
                    wj = wvec[j]
                    wrow_v[e, pl.ds(W - 16, 16)] = jnp.full((16,), wj,
                                                            jnp.float32)
                    wrow_v[e, pl.ds(0, F)] = coord_v[e, :] * wj
                return carry2
            lax.fori_loop(0, BPAD // 16, g_body, 0)

            scs = []
            for j in range(SUBS):
                scs.append(pltpu.async_copy(
                    wrow_v.at[pl.ds(j * Q, Q), :],
                    acc.at[idx_v.at[j]], sem_sc, add=True))
            for d in scs:
                d.wait()
            return carry
        lax.fori_loop(0, NCHUNK, chunk_body, 0)

        plsc.subcore_barrier()
        pltpu.sync_copy(acc.at[pl.ds(r0, ROWS_PER_TILE), :],
                        out_hbm.at[c, pl.ds(r0, ROWS_PER_TILE), :])

    idx2d = indices.reshape(E // Q, Q)
    return sc_kernel(coord_features, idx2d, weights)


def _tc_finish_body(p_ref, emb_ref, o_ref):
    a = p_ref[0] + p_ref[1]
    feat = a[:, 0:F]
    ws = a[:, F:F + 1]
    y = lax.dot_general(feat, emb_ref[...],
                        (((1,), (1,)), ((), ())),
                        preferred_element_type=jnp.float32)
    o_ref[...] = y / ws


def _tc_finish(partials, embedding):
    U = embedding.shape[0]
    R = 2000
    grid = (N // R,)
    return pl.pallas_call(
        _tc_finish_body,
        grid=grid,
        in_specs=[
            pl.BlockSpec((NC, R, W), lambda i: (0, i, 0)),
            pl.BlockSpec((U, F), lambda i: (0, 0)),
        ],
        out_specs=pl.BlockSpec((R, U), lambda i: (i, 0)),
        out_shape=jax.ShapeDtypeStruct((N, U), jnp.float32),
    )(partials, embedding)


def kernel(coord_features, indices, weights, embedding):
    partials = _sc_segment_accumulate(coord_features, indices, weights)
    return _tc_finish(partials, embedding)


# speedup vs baseline: 2.0453x; 1.0293x over previous
import functools

import jax
import jax.numpy as jnp
from jax import lax
from jax.experimental import pallas as pl
from jax.experimental.pallas import tpu as pltpu
from jax.experimental.pallas import tpu_sc as plsc

N = 50000
NPAD = 50048
F = 16
W = 24
NC = 2
NS = 16
NW = NC * NS
B = 1000
BPAD = 1008
Q = 40
SUBS = B // Q
ROWS_PER_TILE = NPAD // NS


def _sc_segment_accumulate(coord_features, indices, weights):
    E = indices.shape[0]
    EP = E // NW
    NCHUNK = EP // B

    mesh = plsc.VectorSubcoreMesh(core_axis_name="c", subcore_axis_name="s")

    @functools.partial(
        pl.kernel,
        mesh=mesh,
        compiler_params=pltpu.CompilerParams(use_tc_tiling_on_sc=False),
        out_type=jax.ShapeDtypeStruct((NC, NPAD, W), jnp.float32),
        scratch_types=[
            pltpu.VMEM((SUBS, Q), jnp.int32),
            pltpu.VMEM((BPAD,), jnp.float32),
            pltpu.VMEM((BPAD, F), jnp.float32),
            pltpu.VMEM((BPAD, W), jnp.float32),
            pltpu.VMEM_SHARED((NPAD, W), jnp.float32),
            pltpu.SemaphoreType.DMA,
            pltpu.SemaphoreType.DMA,
        ],
    )
    def sc_kernel(coord_hbm, idx_hbm, w_hbm, out_hbm,
                  idx_v, w_v, coord_v, wrow_v, acc, sem, sem_sc):
        c = lax.axis_index("c")
        s = lax.axis_index("s")
        wid = c * NS + s

        def zrow(i, carry):
            wrow_v[i, pl.ds(0, 16)] = jnp.zeros((16,), jnp.float32)
            wrow_v[i, pl.ds(W - 16, 16)] = jnp.zeros((16,), jnp.float32)
            return carry
        lax.fori_loop(0, BPAD, zrow, 0)

        r0 = s * ROWS_PER_TILE
        for off, sz in ((0, 1000), (1000, 1000), (2000, 1000), (3000, 128)):
            pltpu.sync_copy(wrow_v.at[pl.ds(0, sz), :],
                            acc.at[pl.ds(r0 + off, sz), :])
        plsc.subcore_barrier()

        def cw_copies(base):
            return [
                pltpu.make_async_copy(
                    w_hbm.at[pl.ds(base, B)], w_v.at[pl.ds(0, B)], sem),
                pltpu.make_async_copy(
                    coord_hbm.at[pl.ds(base, B), :],
                    coord_v.at[pl.ds(0, B), :], sem),
            ]

        def idx_copy(base):
            return pltpu.make_async_copy(
                idx_hbm.at[pl.ds(base // Q, SUBS), :], idx_v, sem)

        base0 = wid * EP
        last = wid * EP + (NCHUNK - 1) * B
        for cp in cw_copies(base0):
            cp.start()
        idx_copy(base0).start()

        def chunk_body(ci, carry):
            base = wid * EP + ci * B
            nxt = jnp.minimum(base + B, last)
            for cp in cw_copies(base):
                cp.wait()
            idx_copy(base).wait()

            def g_body(g, carry2):
                e0 = g * 16
                wvec = w_v[pl.ds(e0, 16)]
                for j in range(16):
                    e = e0 + j
                    wj = wvec[j]
                    wrow_v[e, pl.ds(W - 16, 16)] = jnp.full((16,), wj,
                                                            jnp.float32)
                    wrow_v[e, pl.ds(0, F)] = coord_v[e, :] * wj
                return carry2
            lax.fori_loop(0, BPAD // 16, g_body, 0)

            for cp in cw_copies(nxt):
                cp.start()

            scs = []
            for j in range(SUBS):
                scs.append(pltpu.async_copy(
                    wrow_v.at[pl.ds(j * Q, Q), :],
                    acc.at[idx_v.at[j]], sem_sc, add=True))
            for d in scs:
                d.wait()
            idx_copy(nxt).start()
            return carry
        lax.fori_loop(0, NCHUNK, chunk_body, 0)
        for cp in cw_copies(last):
            cp.wait()
        idx_copy(last).wait()

        plsc.subcore_barrier()
        pltpu.sync_copy(acc.at[pl.ds(r0, ROWS_PER_TILE), :],
                        out_hbm.at[c, pl.ds(r0, ROWS_PER_TILE), :])

    idx2d = indices.reshape(E // Q, Q)
    return sc_kernel(coord_features, idx2d, weights)


def _tc_finish_body(p_ref, emb_ref, o_ref):
    a = p_ref[0] + p_ref[1]
    feat = a[:, 0:F]
    ws = a[:, F:F + 1]
    y = lax.dot_general(feat, emb_ref[...],
                        (((1,), (1,)), ((), ())),
                        preferred_element_type=jnp.float32)
    o_ref[...] = y / ws


def _tc_finish(partials, embedding):
    U = embedding.shape[0]
    R = 2000
    grid = (N // R,)
    return pl.pallas_call(
        _tc_finish_body,
        grid=grid,
        in_specs=[
            pl.BlockSpec((NC, R, W), lambda i: (0, i, 0)),
            pl.BlockSpec((U, F), lambda i: (0, 0)),
        ],
        out_specs=pl.BlockSpec((R, U), lambda i: (i, 0)),
        out_shape=jax.ShapeDtypeStruct((N, U), jnp.float32),
    )(partials, embedding)


def kernel(coord_features, indices, weights, embedding):
    partials = _sc_segment_accumulate(coord_features, indices, weights)
    return _tc_finish(partials, embedding)
